# baseline (speedup 1.0000x reference)
import functools

import jax
import jax.numpy as jnp
from jax import lax
from jax.experimental import pallas as pl
from jax.experimental.pallas import tpu as pltpu
from jax.experimental.pallas import tpu_sc as plsc

VOCAB = 1000000
EMBED = 32
BATCH = 16384

NUM_CORES = 2
NUM_SUBCORES = 16
NUM_WORKERS = NUM_CORES * NUM_SUBCORES
B_PER_W = BATCH // NUM_WORKERS
PASS_ROWS = 256
NPASS = B_PER_W // PASS_ROWS
PGROUPS = PASS_ROWS // 16


def _sc_body(x_hbm, t_hbm, wu_hbm, wv_hbm, out_hbm,
             xidx, tidx, urows, vrows, outv,
             s0, s1, s2, s3, s4, s5, s6, s7):
    usems = [s0, s1, s2, s3]
    vsems = [s4, s5, s6, s7]
    c = lax.axis_index("c")
    s = lax.axis_index("s")
    wid = s * NUM_CORES + c
    base = wid * B_PER_W

    pltpu.sync_copy(x_hbm.at[pl.ds(base, B_PER_W)], xidx)
    pltpu.sync_copy(t_hbm.at[pl.ds(base, B_PER_W)], tidx)

    lanes = lax.iota(jnp.int32, 16)

    def pass_body(p, carry):
        poff = p * PASS_ROWS

        def fire_body(g, carry):
            xv = xidx[pl.ds(poff + g * 16, 16)]
            tv = tidx[pl.ds(poff + g * 16, 16)]
            for j in range(16):
                row = g * 16 + j
                pltpu.async_copy(wu_hbm.at[pl.ds(xv[j], 1), :],
                                 urows.at[pl.ds(row, 1), :], usems[j % 4])
                pltpu.async_copy(wv_hbm.at[pl.ds(tv[j], 1), :],
                                 vrows.at[pl.ds(row, 1), :], vsems[j % 4])
            return carry

        lax.fori_loop(0, PGROUPS, fire_body, 0)

        qr = PASS_ROWS // 4
        for k in range(4):
            pltpu.make_async_copy(wu_hbm.at[pl.ds(0, qr), :],
                                  urows.at[pl.ds(0, qr), :], usems[k]).wait()
            pltpu.make_async_copy(wv_hbm.at[pl.ds(0, qr), :],
                                  vrows.at[pl.ds(0, qr), :], vsems[k]).wait()

        def group_body(g, carry):
            rows = g * 16 + lanes
            acc = jnp.zeros((16,), jnp.float32)
            for j in range(EMBED):
                col = jnp.full((16,), j, jnp.int32)
                uv = plsc.load_gather(urows, [rows, col])
                vv = plsc.load_gather(vrows, [rows, col])
                acc = acc + uv * vv
            outv[pl.ds(poff + g * 16, 16)] = acc
            return carry

        lax.fori_loop(0, PGROUPS, group_body, 0)
        return carry

    lax.fori_loop(0, NPASS, pass_body, 0)

    pltpu.sync_copy(outv, out_hbm.at[pl.ds(base, B_PER_W)])


@jax.jit
def _run(x, t, W_u, W_v):
    mesh = plsc.VectorSubcoreMesh(core_axis_name="c", subcore_axis_name="s")
    kfn = pl.kernel(
        _sc_body,
        out_type=jax.ShapeDtypeStruct((BATCH,), jnp.float32),
        mesh=mesh,
        scratch_types=[
            pltpu.VMEM((B_PER_W,), jnp.int32),
            pltpu.VMEM((B_PER_W,), jnp.int32),
            pltpu.VMEM((PASS_ROWS, EMBED), jnp.float32),
            pltpu.VMEM((PASS_ROWS, EMBED), jnp.float32),
            pltpu.VMEM((B_PER_W,), jnp.float32),
            pltpu.SemaphoreType.DMA,
            pltpu.SemaphoreType.DMA,
            pltpu.SemaphoreType.DMA,
            pltpu.SemaphoreType.DMA,
            pltpu.SemaphoreType.DMA,
            pltpu.SemaphoreType.DMA,
            pltpu.SemaphoreType.DMA,
            pltpu.SemaphoreType.DMA,
        ],
        compiler_params=pltpu.CompilerParams(needs_layout_passes=False),
    )
    return kfn(x, t, W_u, W_v)


def kernel(x, t, W_u, W_v):
    return _run(x.astype(jnp.int32), t.astype(jnp.int32), W_u, W_v)
